# Initial kernel scaffold; baseline (speedup 1.0000x reference)
import functools

import jax
import jax.numpy as jnp
import numpy as np
from jax import lax
from jax.experimental import pallas as pl
from jax.experimental.pallas import tpu as pltpu
from jax.experimental.pallas import tpu_sc as plsc

N = 10000
E = 160000
D_IN = 128
D_OUT = 128
H = 8
DK = 16
D_EDGE = 16

NC = 2
NS = 16
NW = NC * NS
L = 16

C = 128
NCHUNK = E // C
ACC_W = 144
ROWS_PER_SUB = N // NS


def _qkv_body(x_ref, w_ref, b_ref, q_ref, k_ref, v_ref):
    y = jnp.dot(x_ref[...], w_ref[...], preferred_element_type=jnp.float32)
    y = y + b_ref[...]
    q_ref[...] = y[:, 0:128]
    k_ref[...] = y[:, 128:256]
    v_ref[...] = y[:, 256:384]


def _qkv_call(x, w3, b3):
    bn = 1000
    grid = (N // bn,)
    return pl.pallas_call(
        _qkv_body,
        grid=grid,
        in_specs=[
            pl.BlockSpec((bn, D_IN), lambda i: (i, 0)),
            pl.BlockSpec((D_IN, 3 * D_OUT), lambda i: (0, 0)),
            pl.BlockSpec((1, 3 * D_OUT), lambda i: (0, 0)),
        ],
        out_specs=[
            pl.BlockSpec((bn, D_OUT), lambda i: (i, 0)),
            pl.BlockSpec((bn, D_OUT), lambda i: (i, 0)),
            pl.BlockSpec((bn, D_OUT), lambda i: (i, 0)),
        ],
        out_shape=[
            jax.ShapeDtypeStruct((N, D_OUT), jnp.float32),
            jax.ShapeDtypeStruct((N, D_OUT), jnp.float32),
            jax.ShapeDtypeStruct((N, D_OUT), jnp.float32),
        ],
    )(x, w3, b3)


def _ep_body(ea_ref, we_ref, be_ref, ep_ref):
    ep_ref[...] = (
        jnp.dot(ea_ref[...], we_ref[...], preferred_element_type=jnp.float32)
        + be_ref[...]
    )


def _ep_call(edge_attr, We, be):
    bn = 2000
    grid = (E // bn,)
    return pl.pallas_call(
        _ep_body,
        grid=grid,
        in_specs=[
            pl.BlockSpec((bn, D_EDGE), lambda i: (i, 0)),
            pl.BlockSpec((D_EDGE, D_OUT), lambda i: (0, 0)),
            pl.BlockSpec((1, D_OUT), lambda i: (0, 0)),
        ],
        out_specs=pl.BlockSpec((bn, D_OUT), lambda i: (i, 0)),
        out_shape=jax.ShapeDtypeStruct((E, D_OUT), jnp.float32),
    )(edge_attr, We, be)


def _edge_pass_body(q_hbm, k_hbm, v_hbm, ep_hbm, src_hbm, dst_hbm, out_hbm,
                    srcv, dstv, kbuf, vbuf, qbuf, epbuf, obuf,
                    acc, sem_k, sem_v, sem_q, sem_e):
    cid = lax.axis_index("c")
    sid = lax.axis_index("s")
    wid = sid * NC + cid

    zeros16 = jnp.zeros((L,), jnp.float32)

    def _zrow(i, _):
        for jj in range(ACC_W // L):
            obuf[i, pl.ds(jj * L, L)] = zeros16
        return 0
    lax.fori_loop(0, C, _zrow, 0)

    base_row = sid * ROWS_PER_SUB
    for off, nrows in ((0, 128), (128, 128), (256, 128), (384, 128), (512, 113)):
        pltpu.sync_copy(obuf.at[pl.ds(0, nrows)],
                        acc.at[pl.ds(base_row + off, nrows)])
    plsc.subcore_barrier()

    rows_i32 = lax.iota(jnp.int32, L)

    def _chunk(i, _):
        base = (wid + i * NW) * C
        pltpu.sync_copy(src_hbm.at[pl.ds(base, C)], srcv)
        pltpu.sync_copy(dst_hbm.at[pl.ds(base, C)], dstv)
        ck = pltpu.async_copy(k_hbm.at[srcv], kbuf, sem_k)
        cv = pltpu.async_copy(v_hbm.at[srcv], vbuf, sem_v)
        cq = pltpu.async_copy(q_hbm.at[dstv], qbuf, sem_q)
        ce = pltpu.async_copy(ep_hbm.at[pl.ds(base, C)], epbuf, sem_e)
        ck.wait()
        cv.wait()
        cq.wait()
        ce.wait()

        for g in range(C // L):
            erow = rows_i32 + (g * L)
            for h in range(H):
                dbase = h * DK

                def _score(jj, s):
                    d = dbase + jj
                    col = jnp.full((L,), d, jnp.int32)
                    qc = plsc.load_gather(qbuf, [erow, col])
                    kc = plsc.load_gather(kbuf, [erow, col])
                    ec = plsc.load_gather(epbuf, [erow, col])
                    return s + qc * (kc + ec)
                s = lax.fori_loop(0, DK, _score, zeros16)
                ex = jnp.exp(s)
                plsc.store_scatter(
                    obuf, [erow, jnp.full((L,), 128 + h, jnp.int32)], ex)

                def _emit(jj, _):
                    d = dbase + jj
                    col = jnp.full((L,), d, jnp.int32)
                    vc = plsc.load_gather(vbuf, [erow, col])
                    ec = plsc.load_gather(epbuf, [erow, col])
                    plsc.store_scatter(obuf, [erow, col], ex * (vc + ec))
                    return 0
                lax.fori_loop(0, DK, _emit, 0)

        pltpu.sync_copy(obuf, acc.at[dstv], add=True)
        return 0

    n_i = (NCHUNK - 1 - wid) // NW + 1
    lax.fori_loop(0, n_i, _chunk, 0)

    plsc.subcore_barrier()

    for off, nrows in ((0, 128), (128, 128), (256, 128), (384, 128), (512, 113)):
        pltpu.sync_copy(acc.at[pl.ds(base_row + off, nrows)],
                        out_hbm.at[cid, pl.ds(base_row + off, nrows)])


def _edge_pass(q, k, v, ep, src, dst):
    mesh = plsc.VectorSubcoreMesh(core_axis_name="c", subcore_axis_name="s")
    kern = functools.partial(
        pl.kernel,
        mesh=mesh,
        out_type=jax.ShapeDtypeStruct((NC, N, ACC_W), jnp.float32),
        scratch_types=[
            pltpu.VMEM((C,), jnp.int32),
            pltpu.VMEM((C,), jnp.int32),
            pltpu.VMEM((C, D_OUT), jnp.float32),
            pltpu.VMEM((C, D_OUT), jnp.float32),
            pltpu.VMEM((C, D_OUT), jnp.float32),
            pltpu.VMEM((C, D_OUT), jnp.float32),
            pltpu.VMEM((C, ACC_W), jnp.float32),
            pltpu.VMEM_SHARED((N, ACC_W), jnp.float32),
            pltpu.SemaphoreType.DMA,
            pltpu.SemaphoreType.DMA,
            pltpu.SemaphoreType.DMA,
            pltpu.SemaphoreType.DMA,
        ],
    )(_edge_pass_body)
    return kern(q, k, v, ep, src, dst)


def _fin_body(a0_ref, a1_ref, x_ref, wa_ref, ba_ref, dexp_ref, alpha_ref,
              lns_ref, lnb_ref, o_ref):
    a = a0_ref[0] + a1_ref[0]
    num = a[:, 0:128]
    den = a[:, 128:136]
    r = 1.0 / (den + 1e-9)
    rx = jnp.dot(r, dexp_ref[...], preferred_element_type=jnp.float32)
    agg = num * rx
    trans = jnp.dot(agg, wa_ref[...], preferred_element_type=jnp.float32)
    trans = trans + ba_ref[...]
    alpha = alpha_ref[0, 0]
    out = trans * alpha + x_ref[...] * (1.0 - alpha)
    mu = jnp.mean(out, axis=1, keepdims=True)
    cen = out - mu
    var = jnp.mean(cen * cen, axis=1, keepdims=True)
    o_ref[...] = cen * lax.rsqrt(var + 1e-5) * lns_ref[...] + lnb_ref[...]


def _fin_call(acc, x, Wa, ba, dexp, alpha2, lns, lnb):
    bn = 1000
    grid = (N // bn,)
    return pl.pallas_call(
        _fin_body,
        grid=grid,
        in_specs=[
            pl.BlockSpec((1, bn, ACC_W), lambda i: (0, i, 0)),
            pl.BlockSpec((1, bn, ACC_W), lambda i: (1, i, 0)),
            pl.BlockSpec((bn, D_IN), lambda i: (i, 0)),
            pl.BlockSpec((D_OUT, D_OUT), lambda i: (0, 0)),
            pl.BlockSpec((1, D_OUT), lambda i: (0, 0)),
            pl.BlockSpec((H, D_OUT), lambda i: (0, 0)),
            pl.BlockSpec((1, 1), lambda i: (0, 0)),
            pl.BlockSpec((1, D_OUT), lambda i: (0, 0)),
            pl.BlockSpec((1, D_OUT), lambda i: (0, 0)),
        ],
        out_specs=pl.BlockSpec((bn, D_OUT), lambda i: (i, 0)),
        out_shape=jax.ShapeDtypeStruct((N, D_OUT), jnp.float32),
    )(acc, acc, x, Wa, ba, dexp, alpha2, lns, lnb)


_DEXP = np.zeros((H, D_OUT), np.float32)
for _h in range(H):
    _DEXP[_h, _h * DK:(_h + 1) * DK] = 1.0
_DEXP = jnp.asarray(_DEXP)


def kernel(x, edge_attr, Wk, bk, Wq, bq, Wv, bv, Wa, ba, We, be, rel_pri,
           rel_att, rel_msg, skip, ln_scale, ln_bias, edge_index):
    scale = rel_pri / jnp.sqrt(jnp.float32(DK))
    Wq_f = (Wq.reshape(D_IN, H, DK) * scale[None, :, None]).reshape(D_IN, D_OUT)
    Wk_f = jnp.einsum('dhi,hij->dhj', Wk.reshape(D_IN, H, DK), rel_att).reshape(D_IN, D_OUT)
    Wv_f = jnp.einsum('dhi,hij->dhj', Wv.reshape(D_IN, H, DK), rel_msg).reshape(D_IN, D_OUT)
    bq_f = (bq.reshape(H, DK) * scale[:, None]).reshape(D_OUT)
    bk_f = jnp.einsum('hi,hij->hj', bk.reshape(H, DK), rel_att).reshape(D_OUT)
    bv_f = jnp.einsum('hi,hij->hj', bv.reshape(H, DK), rel_msg).reshape(D_OUT)
    w3 = jnp.concatenate([Wq_f, Wk_f, Wv_f], axis=1)
    b3 = jnp.concatenate([bq_f, bk_f, bv_f]).reshape(1, 3 * D_OUT)

    src = edge_index[0]
    dst = edge_index[1]

    q, k, v = _qkv_call(x, w3, b3)
    ep = _ep_call(edge_attr, We, be)

    acc = _edge_pass(q, k, v, ep, src, dst)

    alpha2 = jax.nn.sigmoid(skip).reshape(1, 1)
    out = _fin_call(acc, x, Wa, ba, _DEXP,
                    alpha2, ln_scale.reshape(1, D_OUT), ln_bias.reshape(1, D_OUT))
    return out


# speedup vs baseline: 20.4814x; 20.4814x over previous
import functools

import jax
import jax.numpy as jnp
import numpy as np
from jax import lax
from jax.experimental import pallas as pl
from jax.experimental.pallas import tpu as pltpu
from jax.experimental.pallas import tpu_sc as plsc

N = 10000
E = 160000
D_IN = 128
D_OUT = 128
H = 8
DK = 16
D_EDGE = 16

NC = 2
NS = 16
NW = NC * NS
L = 16

C = 64
NCHUNK = E // C
DEN_W = 16


def _qkv_body(x_ref, w_ref, b_ref, q_ref, k_ref, v_ref):
    y = jnp.dot(x_ref[...], w_ref[...], preferred_element_type=jnp.float32)
    y = y + b_ref[...]
    q_ref[...] = y[:, 0:128]
    k_ref[...] = y[:, 128:256]
    v_ref[...] = y[:, 256:384]


def _qkv_call(x, w3, b3):
    bn = 1000
    grid = (N // bn,)
    return pl.pallas_call(
        _qkv_body,
        grid=grid,
        in_specs=[
            pl.BlockSpec((bn, D_IN), lambda i: (i, 0)),
            pl.BlockSpec((D_IN, 3 * D_OUT), lambda i: (0, 0)),
            pl.BlockSpec((1, 3 * D_OUT), lambda i: (0, 0)),
        ],
        out_specs=[
            pl.BlockSpec((bn, D_OUT), lambda i: (i, 0)),
            pl.BlockSpec((bn, D_OUT), lambda i: (i, 0)),
            pl.BlockSpec((bn, D_OUT), lambda i: (i, 0)),
        ],
        out_shape=[
            jax.ShapeDtypeStruct((N, D_OUT), jnp.float32),
            jax.ShapeDtypeStruct((N, D_OUT), jnp.float32),
            jax.ShapeDtypeStruct((N, D_OUT), jnp.float32),
        ],
    )(x, w3, b3)


def _ep_body(ea_ref, we_ref, be_ref, ep_ref):
    ep_ref[...] = (
        jnp.dot(ea_ref[...], we_ref[...], preferred_element_type=jnp.float32)
        + be_ref[...]
    )


def _ep_call(edge_attr, We, be):
    bn = 2000
    grid = (E // bn,)
    return pl.pallas_call(
        _ep_body,
        grid=grid,
        in_specs=[
            pl.BlockSpec((bn, D_EDGE), lambda i: (i, 0)),
            pl.BlockSpec((D_EDGE, D_OUT), lambda i: (0, 0)),
            pl.BlockSpec((1, D_OUT), lambda i: (0, 0)),
        ],
        out_specs=pl.BlockSpec((bn, D_OUT), lambda i: (i, 0)),
        out_shape=jax.ShapeDtypeStruct((E, D_OUT), jnp.float32),
    )(edge_attr, We, be)


def _edge_pass_body(q_hbm, k_hbm, v_hbm, ep_hbm, src_hbm, dst_hbm,
                    num_hbm, den_hbm,
                    srcv, dstv, kbuf, vbuf, qbuf, epbuf, denbuf, tmp,
                    acc_num, acc_den, sem_k, sem_v, sem_q, sem_e):
    cid = lax.axis_index("c")
    sid = lax.axis_index("s")
    wid = sid * NC + cid

    zeros16 = jnp.zeros((L,), jnp.float32)

    def _zrow(i, _):
        for jj in range(D_OUT // L):
            kbuf[i, pl.ds(jj * L, L)] = zeros16
        denbuf[i, pl.ds(0, L)] = zeros16
        return 0
    lax.fori_loop(0, C, _zrow, 0)
    for h in range(H):
        tmp[h, pl.ds(0, L)] = zeros16
        tmp[h, pl.ds(L, L)] = zeros16

    base_row = sid * 624
    n_init = 39 + jnp.where(sid == NS - 1, 1, 0)

    def _init(i, _):
        pltpu.sync_copy(kbuf.at[pl.ds(0, 16)],
                        acc_num.at[pl.ds(base_row + i * 16, 16)])
        pltpu.sync_copy(denbuf.at[pl.ds(0, 16)],
                        acc_den.at[pl.ds(base_row + i * 16, 16)])
        return 0
    lax.fori_loop(0, n_init, _init, 0)
    plsc.subcore_barrier()

    rows_i32 = lax.iota(jnp.int32, L)

    def _chunk(i, _):
        base = (wid + i * NW) * C
        pltpu.sync_copy(src_hbm.at[pl.ds(base, C)], srcv)
        pltpu.sync_copy(dst_hbm.at[pl.ds(base, C)], dstv)
        ck = pltpu.async_copy(k_hbm.at[srcv], kbuf, sem_k)
        cv = pltpu.async_copy(v_hbm.at[srcv], vbuf, sem_v)
        cq = pltpu.async_copy(q_hbm.at[dstv], qbuf, sem_q)
        ce = pltpu.async_copy(ep_hbm.at[pl.ds(base, C)], epbuf, sem_e)
        ck.wait()
        cv.wait()
        cq.wait()
        ce.wait()

        def _edge(e, _):
            den = zeros16
            r2s = []
            for h in range(H):
                hsl = pl.ds(h * DK, DK)
                ec = epbuf[e, hsl]
                t = qbuf[e, hsl] * (kbuf[e, hsl] + ec)
                tmp[h, pl.ds(0, L)] = t
                r1 = tmp[h, pl.ds(0, L)] + tmp[h, pl.ds(8, L)]
                tmp[h, pl.ds(0, L)] = r1
                r2s.append(tmp[h, pl.ds(0, L)] + tmp[h, pl.ds(4, L)])
            for h in range(H):
                r2 = r2s[h]
                s = ((r2[0] + r2[1]) + (r2[2] + r2[3]))
                ex = jnp.exp(jnp.full((L,), s, jnp.float32))
                den = jnp.where(rows_i32 == h, ex, den)
                hsl = pl.ds(h * DK, DK)
                qbuf[e, hsl] = ex * (vbuf[e, hsl] + epbuf[e, hsl])
            denbuf[e, pl.ds(0, L)] = den
            return 0
        lax.fori_loop(0, C, _edge, 0)

        pltpu.sync_copy(qbuf, acc_num.at[dstv], add=True)
        pltpu.sync_copy(denbuf, acc_den.at[dstv], add=True)
        return 0

    n_i = (NCHUNK - 1 - wid) // NW + 1
    lax.fori_loop(0, n_i, _chunk, 0)

    plsc.subcore_barrier()

    def _readout(i, _):
        pltpu.sync_copy(acc_num.at[pl.ds(base_row + i * 16, 16)],
                        num_hbm.at[cid, pl.ds(base_row + i * 16, 16)])
        pltpu.sync_copy(acc_den.at[pl.ds(base_row + i * 16, 16)],
                        den_hbm.at[cid, pl.ds(base_row + i * 16, 16)])
        return 0
    lax.fori_loop(0, n_init, _readout, 0)


def _edge_pass(q, k, v, ep, src, dst):
    mesh = plsc.VectorSubcoreMesh(core_axis_name="c", subcore_axis_name="s")
    kern = functools.partial(
        pl.kernel,
        mesh=mesh,
        compiler_params=pltpu.CompilerParams(use_tc_tiling_on_sc=False),
        out_type=[
            jax.ShapeDtypeStruct((NC, N, D_OUT), jnp.float32),
            jax.ShapeDtypeStruct((NC, N, DEN_W), jnp.float32),
        ],
        scratch_types=[
            pltpu.VMEM((C,), jnp.int32),
            pltpu.VMEM((C,), jnp.int32),
            pltpu.VMEM((C, D_OUT), jnp.float32),
            pltpu.VMEM((C, D_OUT), jnp.float32),
            pltpu.VMEM((C, D_OUT), jnp.float32),
            pltpu.VMEM((C, D_OUT), jnp.float32),
            pltpu.VMEM((C, DEN_W), jnp.float32),
            pltpu.VMEM((H, 2 * L), jnp.float32),
            pltpu.VMEM_SHARED((N, D_OUT), jnp.float32),
            pltpu.VMEM_SHARED((N, DEN_W), jnp.float32),
            pltpu.SemaphoreType.DMA,
            pltpu.SemaphoreType.DMA,
            pltpu.SemaphoreType.DMA,
            pltpu.SemaphoreType.DMA,
        ],
    )(_edge_pass_body)
    return kern(q, k, v, ep, src, dst)


def _fin_body(n0_ref, n1_ref, d0_ref, d1_ref, x_ref, wa_ref, ba_ref, dexp_ref,
              alpha_ref, lns_ref, lnb_ref, o_ref):
    num = n0_ref[0] + n1_ref[0]
    den = (d0_ref[0] + d1_ref[0])[:, 0:H]
    r = 1.0 / (den + 1e-9)
    rx = jnp.dot(r, dexp_ref[...], preferred_element_type=jnp.float32)
    agg = num * rx
    trans = jnp.dot(agg, wa_ref[...], preferred_element_type=jnp.float32)
    trans = trans + ba_ref[...]
    alpha = alpha_ref[0, 0]
    out = trans * alpha + x_ref[...] * (1.0 - alpha)
    mu = jnp.mean(out, axis=1, keepdims=True)
    cen = out - mu
    var = jnp.mean(cen * cen, axis=1, keepdims=True)
    o_ref[...] = cen * lax.rsqrt(var + 1e-5) * lns_ref[...] + lnb_ref[...]


def _fin_call(num, den, x, Wa, ba, dexp, alpha2, lns, lnb):
    bn = 1000
    grid = (N // bn,)
    return pl.pallas_call(
        _fin_body,
        grid=grid,
        in_specs=[
            pl.BlockSpec((1, bn, D_OUT), lambda i: (0, i, 0)),
            pl.BlockSpec((1, bn, D_OUT), lambda i: (1, i, 0)),
            pl.BlockSpec((1, bn, DEN_W), lambda i: (0, i, 0)),
            pl.BlockSpec((1, bn, DEN_W), lambda i: (1, i, 0)),
            pl.BlockSpec((bn, D_IN), lambda i: (i, 0)),
            pl.BlockSpec((D_OUT, D_OUT), lambda i: (0, 0)),
            pl.BlockSpec((1, D_OUT), lambda i: (0, 0)),
            pl.BlockSpec((H, D_OUT), lambda i: (0, 0)),
            pl.BlockSpec((1, 1), lambda i: (0, 0)),
            pl.BlockSpec((1, D_OUT), lambda i: (0, 0)),
            pl.BlockSpec((1, D_OUT), lambda i: (0, 0)),
        ],
        out_specs=pl.BlockSpec((bn, D_OUT), lambda i: (i, 0)),
        out_shape=jax.ShapeDtypeStruct((N, D_OUT), jnp.float32),
    )(num, num, den, den, x, Wa, ba, dexp, alpha2, lns, lnb)


_DEXP = np.zeros((H, D_OUT), np.float32)
for _h in range(H):
    _DEXP[_h, _h * DK:(_h + 1) * DK] = 1.0


def kernel(x, edge_attr, Wk, bk, Wq, bq, Wv, bv, Wa, ba, We, be, rel_pri,
           rel_att, rel_msg, skip, ln_scale, ln_bias, edge_index):
    scale = rel_pri / jnp.sqrt(jnp.float32(DK))
    Wq_f = (Wq.reshape(D_IN, H, DK) * scale[None, :, None]).reshape(D_IN, D_OUT)
    Wk_f = jnp.einsum('dhi,hij->dhj', Wk.reshape(D_IN, H, DK), rel_att).reshape(D_IN, D_OUT)
    Wv_f = jnp.einsum('dhi,hij->dhj', Wv.reshape(D_IN, H, DK), rel_msg).reshape(D_IN, D_OUT)
    bq_f = (bq.reshape(H, DK) * scale[:, None]).reshape(D_OUT)
    bk_f = jnp.einsum('hi,hij->hj', bk.reshape(H, DK), rel_att).reshape(D_OUT)
    bv_f = jnp.einsum('hi,hij->hj', bv.reshape(H, DK), rel_msg).reshape(D_OUT)
    w3 = jnp.concatenate([Wq_f, Wk_f, Wv_f], axis=1)
    b3 = jnp.concatenate([bq_f, bk_f, bv_f]).reshape(1, 3 * D_OUT)

    src = edge_index[0]
    dst = edge_index[1]

    q, k, v = _qkv_call(x, w3, b3)
    ep = _ep_call(edge_attr, We, be.reshape(1, D_OUT))

    num, den = _edge_pass(q, k, v, ep, src, dst)

    alpha2 = jax.nn.sigmoid(skip).reshape(1, 1)
    out = _fin_call(num, den, x, Wa, ba.reshape(1, D_OUT), jnp.asarray(_DEXP),
                    alpha2, ln_scale.reshape(1, D_OUT), ln_bias.reshape(1, D_OUT))
    return out


# speedup vs baseline: 23.5498x; 1.1498x over previous
import functools

import jax
import jax.numpy as jnp
import numpy as np
from jax import lax
from jax.experimental import pallas as pl
from jax.experimental.pallas import tpu as pltpu
from jax.experimental.pallas import tpu_sc as plsc

N = 10000
E = 160000
D_IN = 128
D_OUT = 128
H = 8
DK = 16
D_EDGE = 16

NC = 2
NS = 16
NW = NC * NS
L = 16

C = 32
NCHUNK = E // C
G = 8
NSUP = NCHUNK // G
DEN_W = 16


def _qkv_body(x_ref, w_ref, b_ref, q_ref, kv_ref):
    y = jnp.dot(x_ref[...], w_ref[...], preferred_element_type=jnp.float32)
    y = y + b_ref[...]
    q_ref[...] = y[:, 0:128]
    kv_ref[...] = y[:, 128:384]


def _qkv_call(x, w3, b3):
    bn = 1000
    grid = (N // bn,)
    return pl.pallas_call(
        _qkv_body,
        grid=grid,
        in_specs=[
            pl.BlockSpec((bn, D_IN), lambda i: (i, 0)),
            pl.BlockSpec((D_IN, 3 * D_OUT), lambda i: (0, 0)),
            pl.BlockSpec((1, 3 * D_OUT), lambda i: (0, 0)),
        ],
        out_specs=[
            pl.BlockSpec((bn, D_OUT), lambda i: (i, 0)),
            pl.BlockSpec((bn, 2 * D_OUT), lambda i: (i, 0)),
        ],
        out_shape=[
            jax.ShapeDtypeStruct((N, D_OUT), jnp.float32),
            jax.ShapeDtypeStruct((N, 2 * D_OUT), jnp.float32),
        ],
    )(x, w3, b3)


def _ep_body(ea_ref, we_ref, be_ref, ep_ref):
    ep_ref[...] = (
        jnp.dot(ea_ref[...], we_ref[...], preferred_element_type=jnp.float32)
        + be_ref[...]
    )


def _ep_call(edge_attr, We, be):
    bn = 2000
    grid = (E // bn,)
    return pl.pallas_call(
        _ep_body,
        grid=grid,
        in_specs=[
            pl.BlockSpec((bn, D_EDGE), lambda i: (i, 0)),
            pl.BlockSpec((D_EDGE, D_OUT), lambda i: (0, 0)),
            pl.BlockSpec((1, D_OUT), lambda i: (0, 0)),
        ],
        out_specs=pl.BlockSpec((bn, D_OUT), lambda i: (i, 0)),
        out_shape=jax.ShapeDtypeStruct((E, D_OUT), jnp.float32),
    )(edge_attr, We, be)


def _edge_pass_body(q_hbm, kv_hbm, ep_hbm, src_hbm, dst_hbm,
                    num_hbm, den_hbm,
                    si2, di2, kvb0, kvb1, qb0, qb1, eb0, eb1, db0, db1, tmp,
                    acc_num, acc_den,
                    skv0, skv1, sq0, sq1, se0, se1):
    cid = lax.axis_index("c")
    sid = lax.axis_index("s")
    wid = sid * NC + cid
    kvb = [kvb0, kvb1]
    qb = [qb0, qb1]
    eb = [eb0, eb1]
    db = [db0, db1]
    skv = [skv0, skv1]
    sq = [sq0, sq1]
    se = [se0, se1]

    zeros16 = jnp.zeros((L,), jnp.float32)

    def _z(i, _):
        for jj in range(D_OUT // L):
            qb0[i, pl.ds(jj * L, L)] = zeros16
        db0[i, pl.ds(0, L)] = zeros16
        return 0
    lax.fori_loop(0, 16, _z, 0)
    for h in range(H):
        tmp[h, pl.ds(0, L)] = zeros16
        tmp[h, pl.ds(L, L)] = zeros16

    base_row = sid * 624
    n_init = 39 + jnp.where(sid == NS - 1, 1, 0)

    def _init(i, _):
        pltpu.sync_copy(qb0.at[pl.ds(0, 16)],
                        acc_num.at[pl.ds(base_row + i * 16, 16)])
        pltpu.sync_copy(db0.at[pl.ds(0, 16)],
                        acc_den.at[pl.ds(base_row + i * 16, 16)])
        return 0
    lax.fori_loop(0, n_init, _init, 0)
    plsc.subcore_barrier()

    rows_i32 = lax.iota(jnp.int32, L)
    n_sup = (NSUP - 1 - wid) // NW + 1

    def idx_load(S):
        pltpu.sync_copy(src_hbm.at[pl.ds(S * G, G)], si2)
        pltpu.sync_copy(dst_hbm.at[pl.ds(S * G, G)], di2)

    def g_issue(p, j, base):
        return [
            pltpu.async_copy(kv_hbm.at[si2.at[j]], kvb[p], skv[p]),
            pltpu.async_copy(q_hbm.at[di2.at[j]], qb[p], sq[p]),
            pltpu.async_copy(ep_hbm.at[pl.ds(base, C)], eb[p], se[p]),
        ]

    def compute(p, lo, hi):
        kvb_p, qb_p, eb_p, db_p = kvb[p], qb[p], eb[p], db[p]

        def _edge(e, _):
            den = zeros16
            r2s = []
            ws = []
            for h in range(H):
                hsl = pl.ds(h * DK, DK)
                ec = eb_p[e, hsl]
                t = qb_p[e, hsl] * (kvb_p[e, hsl] + ec)
                ws.append(kvb_p[e, pl.ds(D_OUT + h * DK, DK)] + ec)
                tmp[h, pl.ds(0, L)] = t
                r1 = t + tmp[h, pl.ds(8, L)]
                tmp[h, pl.ds(0, L)] = r1
                r2s.append(r1 + tmp[h, pl.ds(4, L)])
            for h in range(H):
                r2 = r2s[h]
                s = ((r2[0] + r2[1]) + (r2[2] + r2[3]))
                ex = jnp.exp(jnp.full((L,), s, jnp.float32))
                den = jnp.where(rows_i32 == h, ex, den)
                qb_p[e, pl.ds(h * DK, DK)] = ex * ws[h]
            db_p[e, pl.ds(0, L)] = den
            return 0
        lax.fori_loop(lo, hi, _edge, 0)

    def _super(i, _):
        S = wid + i * NW
        base_s = S * (G * C)
        idx_load(S)
        handles = [None, None]
        handles[0] = g_issue(0, 0, base_s)
        for j in range(G):
            p = j % 2
            if j < G - 1:
                handles[1 - p] = g_issue(1 - p, j + 1, base_s + (j + 1) * C)
            for hnd in handles[p]:
                hnd.wait()
            compute(p, 0, C)
            pltpu.sync_copy(qb[p], acc_num.at[di2.at[j]], add=True)
            pltpu.sync_copy(db[p], acc_den.at[di2.at[j]], add=True)
        return 0
    lax.fori_loop(0, n_sup, _super, 0)

    plsc.subcore_barrier()

    def _readout(i, _):
        pltpu.sync_copy(acc_num.at[pl.ds(base_row + i * 16, 16)],
                        num_hbm.at[cid, pl.ds(base_row + i * 16, 16)])
        pltpu.sync_copy(acc_den.at[pl.ds(base_row + i * 16, 16)],
                        den_hbm.at[cid, pl.ds(base_row + i * 16, 16)])
        return 0
    lax.fori_loop(0, n_init, _readout, 0)


def _edge_pass(q, kv, ep, src2d, dst2d):
    mesh = plsc.VectorSubcoreMesh(core_axis_name="c", subcore_axis_name="s")
    kern = functools.partial(
        pl.kernel,
        mesh=mesh,
        compiler_params=pltpu.CompilerParams(use_tc_tiling_on_sc=False),
        out_type=[
            jax.ShapeDtypeStruct((NC, N, D_OUT), jnp.float32),
            jax.ShapeDtypeStruct((NC, N, DEN_W), jnp.float32),
        ],
        scratch_types=[
            pltpu.VMEM((G, C), jnp.int32),
            pltpu.VMEM((G, C), jnp.int32),
            pltpu.VMEM((C, 2 * D_OUT), jnp.float32),
            pltpu.VMEM((C, 2 * D_OUT), jnp.float32),
            pltpu.VMEM((C, D_OUT), jnp.float32),
            pltpu.VMEM((C, D_OUT), jnp.float32),
            pltpu.VMEM((C, D_OUT), jnp.float32),
            pltpu.VMEM((C, D_OUT), jnp.float32),
            pltpu.VMEM((C, DEN_W), jnp.float32),
            pltpu.VMEM((C, DEN_W), jnp.float32),
            pltpu.VMEM((H, 2 * L), jnp.float32),
            pltpu.VMEM_SHARED((N, D_OUT), jnp.float32),
            pltpu.VMEM_SHARED((N, DEN_W), jnp.float32),
        ] + [pltpu.SemaphoreType.DMA] * 6,
    )(_edge_pass_body)
    return kern(q, kv, ep, src2d, dst2d)


def _fin_body(n0_ref, n1_ref, d0_ref, d1_ref, x_ref, wa_ref, ba_ref, dexp_ref,
              alpha_ref, lns_ref, lnb_ref, o_ref):
    num = n0_ref[0] + n1_ref[0]
    den = (d0_ref[0] + d1_ref[0])[:, 0:H]
    r = 1.0 / (den + 1e-9)
    rx = jnp.dot(r, dexp_ref[...], preferred_element_type=jnp.float32)
    agg = num * rx
    trans = jnp.dot(agg, wa_ref[...], preferred_element_type=jnp.float32)
    trans = trans + ba_ref[...]
    alpha = alpha_ref[0, 0]
    out = trans * alpha + x_ref[...] * (1.0 - alpha)
    mu = jnp.mean(out, axis=1, keepdims=True)
    cen = out - mu
    var = jnp.mean(cen * cen, axis=1, keepdims=True)
    o_ref[...] = cen * lax.rsqrt(var + 1e-5) * lns_ref[...] + lnb_ref[...]


def _fin_call(num, den, x, Wa, ba, dexp, alpha2, lns, lnb):
    bn = 1000
    grid = (N // bn,)
    return pl.pallas_call(
        _fin_body,
        grid=grid,
        in_specs=[
            pl.BlockSpec((1, bn, D_OUT), lambda i: (0, i, 0)),
            pl.BlockSpec((1, bn, D_OUT), lambda i: (1, i, 0)),
            pl.BlockSpec((1, bn, DEN_W), lambda i: (0, i, 0)),
            pl.BlockSpec((1, bn, DEN_W), lambda i: (1, i, 0)),
            pl.BlockSpec((bn, D_IN), lambda i: (i, 0)),
            pl.BlockSpec((D_OUT, D_OUT), lambda i: (0, 0)),
            pl.BlockSpec((1, D_OUT), lambda i: (0, 0)),
            pl.BlockSpec((H, D_OUT), lambda i: (0, 0)),
            pl.BlockSpec((1, 1), lambda i: (0, 0)),
            pl.BlockSpec((1, D_OUT), lambda i: (0, 0)),
            pl.BlockSpec((1, D_OUT), lambda i: (0, 0)),
        ],
        out_specs=pl.BlockSpec((bn, D_OUT), lambda i: (i, 0)),
        out_shape=jax.ShapeDtypeStruct((N, D_OUT), jnp.float32),
    )(num, num, den, den, x, Wa, ba, dexp, alpha2, lns, lnb)


_DEXP = np.zeros((H, D_OUT), np.float32)
for _h in range(H):
    _DEXP[_h, _h * DK:(_h + 1) * DK] = 1.0


def kernel(x, edge_attr, Wk, bk, Wq, bq, Wv, bv, Wa, ba, We, be, rel_pri,
           rel_att, rel_msg, skip, ln_scale, ln_bias, edge_index):
    scale = rel_pri / jnp.sqrt(jnp.float32(DK))
    Wq_f = (Wq.reshape(D_IN, H, DK) * scale[None, :, None]).reshape(D_IN, D_OUT)
    Wk_f = jnp.einsum('dhi,hij->dhj', Wk.reshape(D_IN, H, DK), rel_att).reshape(D_IN, D_OUT)
    Wv_f = jnp.einsum('dhi,hij->dhj', Wv.reshape(D_IN, H, DK), rel_msg).reshape(D_IN, D_OUT)
    bq_f = (bq.reshape(H, DK) * scale[:, None]).reshape(D_OUT)
    bk_f = jnp.einsum('hi,hij->hj', bk.reshape(H, DK), rel_att).reshape(D_OUT)
    bv_f = jnp.einsum('hi,hij->hj', bv.reshape(H, DK), rel_msg).reshape(D_OUT)
    w3 = jnp.concatenate([Wq_f, Wk_f, Wv_f], axis=1)
    b3 = jnp.concatenate([bq_f, bk_f, bv_f]).reshape(1, 3 * D_OUT)

    src = edge_index[0]
    dst = edge_index[1]

    q, kv = _qkv_call(x, w3, b3)
    ep = _ep_call(edge_attr, We, be.reshape(1, D_OUT))

    num, den = _edge_pass(q, kv, ep,
                          src.reshape(NCHUNK, C), dst.reshape(NCHUNK, C))

    alpha2 = jax.nn.sigmoid(skip).reshape(1, 1)
    out = _fin_call(num, den, x, Wa, ba.reshape(1, D_OUT), jnp.asarray(_DEXP),
                    alpha2, ln_scale.reshape(1, D_OUT), ln_bias.reshape(1, D_OUT))
    return out
